# baseline (speedup 1.0000x reference)
import functools

import jax
import jax.numpy as jnp
from jax.experimental import pallas as pl

_N = 10000
_E = 160000
_DH = 512
_BLK = 400


def _layer_body(h_ref, r1_ref, r2_ref, w0_ref, w1_ref, w2_ref, b_ref, norm_ref,
                out_ref, xs_ref):
    acc = jnp.dot(h_ref[...], w0_ref[...], preferred_element_type=jnp.float32)
    acc = acc + jnp.dot(r1_ref[...], w1_ref[...], preferred_element_type=jnp.float32)
    acc = acc + jnp.dot(r2_ref[...], w2_ref[...], preferred_element_type=jnp.float32)
    acc = acc + b_ref[...]
    out = jnp.maximum(acc, 0.0)
    out_ref[...] = out
    xs_ref[...] = out * norm_ref[...]


def _tag_layer(h, r1, r2, W, b, norm):
    d_in = h.shape[1]
    w0 = W[:d_in]
    w1 = W[d_in:2 * d_in]
    w2 = W[2 * d_in:]
    b2d = b.reshape(1, _DH)
    grid = _N // _BLK
    out, xs = pl.pallas_call(
        _layer_body,
        grid=(grid,),
        in_specs=[
            pl.BlockSpec((_BLK, d_in), lambda i: (i, 0)),
            pl.BlockSpec((_BLK, d_in), lambda i: (i, 0)),
            pl.BlockSpec((_BLK, d_in), lambda i: (i, 0)),
            pl.BlockSpec((d_in, _DH), lambda i: (0, 0)),
            pl.BlockSpec((d_in, _DH), lambda i: (0, 0)),
            pl.BlockSpec((d_in, _DH), lambda i: (0, 0)),
            pl.BlockSpec((1, _DH), lambda i: (0, 0)),
            pl.BlockSpec((_BLK, 1), lambda i: (i, 0)),
        ],
        out_specs=[
            pl.BlockSpec((_BLK, _DH), lambda i: (i, 0)),
            pl.BlockSpec((_BLK, _DH), lambda i: (i, 0)),
        ],
        out_shape=[
            jax.ShapeDtypeStruct((_N, _DH), jnp.float32),
            jax.ShapeDtypeStruct((_N, _DH), jnp.float32),
        ],
    )(h, r1, r2, w0, w1, w2, b2d, norm)
    return out, xs


def _pool_body(h_ref, wc_ref, bc_ref, hg_ref, lg_ref):
    i = pl.program_id(0)

    @pl.when(i == 0)
    def _():
        hg_ref[...] = jnp.zeros_like(hg_ref)
        lg_ref[...] = jnp.zeros_like(lg_ref)

    hg_ref[...] += jnp.sum(h_ref[...], axis=0, keepdims=True)

    @pl.when(i == pl.num_programs(0) - 1)
    def _():
        hg = hg_ref[...] / _N
        hg_ref[...] = hg
        lg_ref[...] = jnp.dot(hg, wc_ref[...],
                              preferred_element_type=jnp.float32) + bc_ref[...]


def _pool(h, Wc, bc):
    grid = _N // _BLK
    hg, lg = pl.pallas_call(
        _pool_body,
        grid=(grid,),
        in_specs=[
            pl.BlockSpec((_BLK, _DH), lambda i: (i, 0)),
            pl.BlockSpec((_DH, 16), lambda i: (0, 0)),
            pl.BlockSpec((1, 16), lambda i: (0, 0)),
        ],
        out_specs=[
            pl.BlockSpec((1, _DH), lambda i: (0, 0)),
            pl.BlockSpec((1, 16), lambda i: (0, 0)),
        ],
        out_shape=[
            jax.ShapeDtypeStruct((1, _DH), jnp.float32),
            jax.ShapeDtypeStruct((1, 16), jnp.float32),
        ],
    )(h, Wc, bc.reshape(1, 16))
    return hg, lg


def kernel(x, W1, b1, W2, b2, W3, b3, W4, b4, W5, b5, W6, b6, W7, b7, Wc, bc,
           edge_index):
    src = edge_index[0]
    dst = edge_index[1]
    deg = jnp.zeros((_N,), jnp.float32).at[dst].add(1.0)
    norm = jnp.power(jnp.clip(deg, 1.0, None), -0.5)[:, None]

    def prop(xs):
        return jnp.zeros((_N, xs.shape[1]), xs.dtype).at[dst].add(xs[src])

    h = x
    xs = h * norm
    params = [(W1, b1), (W2, b2), (W3, b3), (W4, b4), (W5, b5), (W6, b6),
              (W7, b7)]
    for (W, b) in params:
        agg1 = prop(xs)
        r1 = agg1 * norm
        agg2 = prop(r1 * norm)
        r2 = agg2 * norm
        h, xs = _tag_layer(h, r1, r2, W, b, norm)
    return _pool(h, Wc, bc)


# speedup vs baseline: 1.8401x; 1.8401x over previous
import functools

import jax
import jax.numpy as jnp
from jax import lax
from jax.experimental import pallas as pl
from jax.experimental.pallas import tpu as pltpu
from jax.experimental.pallas import tpu_sc as plsc

_N = 10000
_E = 160000
_DH = 512
_NCLS = 16

_NC = 2
_NS = 16
_NT = _NC * _NS
_EPT = 5120
_EPAD = _EPT * _NT
_NBATCH = _EPT // 128
_NPAD = 10240
_RPT = _NPAD // _NS
_BLK = 400


def _sc_mesh():
    return plsc.VectorSubcoreMesh(core_axis_name="c", subcore_axis_name="s",
                                  num_cores=_NC, num_subcores=_NS)


def _make_prop_body(nchunk):
    def body(*refs):
        xs = refs[:nchunk]
        src_hbm = refs[nchunk]
        dst_hbm = refs[nchunk + 1]
        outs = refs[nchunk + 2:2 * nchunk + 2]
        (src_v, dst_v, buf_a, buf_b, zbuf, spmem, sem_a, sem_b) = \
            refs[2 * nchunk + 2:]
        c = lax.axis_index("c")
        s = lax.axis_index("s")
        wid = c * _NS + s
        pltpu.sync_copy(src_hbm.at[pl.ds(wid * _NBATCH, _NBATCH)], src_v)
        pltpu.sync_copy(dst_hbm.at[pl.ds(wid * _NBATCH, _NBATCH)], dst_v)

        def fill_zero(i, _):
            for j in range(8):
                zbuf[i, pl.ds(j * 16, 16)] = jnp.zeros((16,), jnp.float32)
            return 0

        lax.fori_loop(0, 40, fill_zero, 0)

        for ci in range(nchunk):
            for k in range(16):
                pltpu.sync_copy(zbuf, spmem.at[pl.ds(s * _RPT + k * 40, 40)])
            plsc.subcore_barrier()
            xs_c = xs[ci]

            def start_gather(b, buf, sem):
                pltpu.async_copy(xs_c.at[src_v.at[b]], buf, sem)

            def wait_gather(buf, sem):
                pltpu.make_async_copy(xs_c.at[src_v.at[0]], buf, sem).wait()

            def scat(b, buf):
                pltpu.sync_copy(buf, spmem.at[dst_v.at[b]], add=True)

            start_gather(0, buf_a, sem_a)

            def grp(g, _):
                b0 = 2 * g
                start_gather(b0 + 1, buf_b, sem_b)
                wait_gather(buf_a, sem_a)
                scat(b0, buf_a)

                @pl.when(b0 + 2 < _NBATCH)
                def _():
                    start_gather(b0 + 2, buf_a, sem_a)

                wait_gather(buf_b, sem_b)
                scat(b0 + 1, buf_b)
                return 0

            lax.fori_loop(0, _NBATCH // 2, grp, 0)
            plsc.subcore_barrier()
            pltpu.sync_copy(spmem.at[pl.ds(s * _RPT, _RPT)],
                            outs[ci].at[pl.ds(c * _NPAD + s * _RPT, _RPT)])

    return body


@functools.cache
def _prop_call(nchunk):
    return pl.kernel(
        _make_prop_body(nchunk),
        out_type=[jax.ShapeDtypeStruct((_NC * _NPAD, 128), jnp.float32)
                  for _ in range(nchunk)],
        mesh=_sc_mesh(),
        scratch_types=[
            pltpu.VMEM((_NBATCH, 128), jnp.int32),
            pltpu.VMEM((_NBATCH, 128), jnp.int32),
            pltpu.VMEM((128, 128), jnp.float32),
            pltpu.VMEM((128, 128), jnp.float32),
            pltpu.VMEM((40, 128), jnp.float32),
            pltpu.VMEM_SHARED((_NPAD, 128), jnp.float32),
            pltpu.SemaphoreType.DMA,
            pltpu.SemaphoreType.DMA,
        ],
    )


def _norm_body(degp_ref, x_ref, norm_ref, xs0_ref, xs1_ref):
    d = degp_ref[...]
    deg = d[0, :, 0:1] + d[1, :, 0:1]
    nrm = lax.rsqrt(jnp.maximum(deg, 1.0))
    norm_ref[...] = nrm
    xv = x_ref[...]
    xs0_ref[...] = xv[:, 0:128] * nrm
    xs1_ref[...] = xv[:, 128:256] * nrm


def _norm_call(degp, x):
    grid = _N // _BLK
    return pl.pallas_call(
        _norm_body,
        grid=(grid,),
        in_specs=[
            pl.BlockSpec((_NC, _BLK, 128), lambda i: (0, i, 0)),
            pl.BlockSpec((_BLK, 256), lambda i: (i, 0)),
        ],
        out_specs=[
            pl.BlockSpec((_BLK, 1), lambda i: (i, 0)),
            pl.BlockSpec((_BLK, 128), lambda i: (i, 0)),
            pl.BlockSpec((_BLK, 128), lambda i: (i, 0)),
        ],
        out_shape=[
            jax.ShapeDtypeStruct((_N, 1), jnp.float32),
            jax.ShapeDtypeStruct((_N, 128), jnp.float32),
            jax.ShapeDtypeStruct((_N, 128), jnp.float32),
        ],
    )(degp, x)


def _make_mid_body(nchunk):
    def body(*refs):
        aggs = refs[:nchunk]
        norm_ref = refs[nchunk]
        r1s = refs[nchunk + 1:2 * nchunk + 1]
        xs2s = refs[2 * nchunk + 1:]
        nrm = norm_ref[...]
        for ci in range(nchunk):
            a = aggs[ci][...]
            r1 = (a[0] + a[1]) * nrm
            r1s[ci][...] = r1
            xs2s[ci][...] = r1 * nrm
    return body


def _mid_call(aggs, norm):
    nchunk = len(aggs)
    grid = _N // _BLK
    out = pl.pallas_call(
        _make_mid_body(nchunk),
        grid=(grid,),
        in_specs=(
            [pl.BlockSpec((_NC, _BLK, 128), lambda i: (0, i, 0))
             for _ in range(nchunk)]
            + [pl.BlockSpec((_BLK, 1), lambda i: (i, 0))]
        ),
        out_specs=[pl.BlockSpec((_BLK, 128), lambda i: (i, 0))
                   for _ in range(2 * nchunk)],
        out_shape=[jax.ShapeDtypeStruct((_N, 128), jnp.float32)
                   for _ in range(2 * nchunk)],
    )(*aggs, norm)
    return out[:nchunk], out[nchunk:]


def _make_layer_body(nchunk, emit_xs):
    def body(*refs):
        i = 0
        h_ref = refs[i]; i += 1
        r1s = refs[i:i + nchunk]; i += nchunk
        aggs2 = refs[i:i + nchunk]; i += nchunk
        w0_ref = refs[i]; i += 1
        w1s = refs[i:i + nchunk]; i += nchunk
        w2s = refs[i:i + nchunk]; i += nchunk
        b_ref = refs[i]; i += 1
        norm_ref = refs[i]; i += 1
        out_ref = refs[i]; i += 1
        xs_refs = refs[i:]
        nrm = norm_ref[...]
        acc = jnp.dot(h_ref[...], w0_ref[...],
                      preferred_element_type=jnp.float32)
        for ci in range(nchunk):
            acc += jnp.dot(r1s[ci][...], w1s[ci][...],
                           preferred_element_type=jnp.float32)
            a = aggs2[ci][...]
            r2 = (a[0] + a[1]) * nrm
            acc += jnp.dot(r2, w2s[ci][...],
                           preferred_element_type=jnp.float32)
        acc += b_ref[...]
        out = jnp.maximum(acc, 0.0)
        out_ref[...] = out
        if emit_xs:
            for co in range(4):
                xs_refs[co][...] = out[:, co * 128:(co + 1) * 128] * nrm
    return body


def _layer_call(h, r1s, aggs2, W, b, norm, emit_xs):
    d_in = h.shape[1]
    nchunk = d_in // 128
    w0 = W[:d_in]
    w1s = [W[d_in + ci * 128:d_in + (ci + 1) * 128] for ci in range(nchunk)]
    w2s = [W[2 * d_in + ci * 128:2 * d_in + (ci + 1) * 128]
           for ci in range(nchunk)]
    grid = _N // _BLK
    n_xs = 4 if emit_xs else 0
    out = pl.pallas_call(
        _make_layer_body(nchunk, emit_xs),
        grid=(grid,),
        in_specs=(
            [pl.BlockSpec((_BLK, d_in), lambda i: (i, 0))]
            + [pl.BlockSpec((_BLK, 128), lambda i: (i, 0))
               for _ in range(nchunk)]
            + [pl.BlockSpec((_NC, _BLK, 128), lambda i: (0, i, 0))
               for _ in range(nchunk)]
            + [pl.BlockSpec((d_in, _DH), lambda i: (0, 0))]
            + [pl.BlockSpec((128, _DH), lambda i: (0, 0))
               for _ in range(2 * nchunk)]
            + [pl.BlockSpec((1, _DH), lambda i: (0, 0)),
               pl.BlockSpec((_BLK, 1), lambda i: (i, 0))]
        ),
        out_specs=(
            [pl.BlockSpec((_BLK, _DH), lambda i: (i, 0))]
            + [pl.BlockSpec((_BLK, 128), lambda i: (i, 0))
               for _ in range(n_xs)]
        ),
        out_shape=(
            [jax.ShapeDtypeStruct((_N, _DH), jnp.float32)]
            + [jax.ShapeDtypeStruct((_N, 128), jnp.float32)
               for _ in range(n_xs)]
        ),
    )(h, *r1s, *aggs2, w0, *w1s, *w2s, b.reshape(1, _DH), norm)
    if emit_xs:
        return out[0], out[1:]
    return out[0], None


def _pool_body(h_ref, wc_ref, bc_ref, hg_ref, lg_ref):
    i = pl.program_id(0)

    @pl.when(i == 0)
    def _():
        hg_ref[...] = jnp.zeros_like(hg_ref)
        lg_ref[...] = jnp.zeros_like(lg_ref)

    hg_ref[...] += jnp.sum(h_ref[...], axis=0, keepdims=True)

    @pl.when(i == pl.num_programs(0) - 1)
    def _():
        hg = hg_ref[...] / _N
        hg_ref[...] = hg
        lg_ref[...] = jnp.dot(hg, wc_ref[...],
                              preferred_element_type=jnp.float32) + bc_ref[...]


def _pool(h, Wc, bc):
    grid = _N // _BLK
    return pl.pallas_call(
        _pool_body,
        grid=(grid,),
        in_specs=[
            pl.BlockSpec((_BLK, _DH), lambda i: (i, 0)),
            pl.BlockSpec((_DH, _NCLS), lambda i: (0, 0)),
            pl.BlockSpec((1, _NCLS), lambda i: (0, 0)),
        ],
        out_specs=[
            pl.BlockSpec((1, _DH), lambda i: (0, 0)),
            pl.BlockSpec((1, _NCLS), lambda i: (0, 0)),
        ],
        out_shape=[
            jax.ShapeDtypeStruct((1, _DH), jnp.float32),
            jax.ShapeDtypeStruct((1, _NCLS), jnp.float32),
        ],
    )(h, Wc, bc.reshape(1, _NCLS))


def kernel(x, W1, b1, W2, b2, W3, b3, W4, b4, W5, b5, W6, b6, W7, b7, Wc, bc,
           edge_index):
    src = edge_index[0].astype(jnp.int32)
    dst = edge_index[1].astype(jnp.int32)
    npad = _EPAD - _E
    src2d = jnp.concatenate(
        [src, jnp.zeros((npad,), jnp.int32)]).reshape(_EPAD // 128, 128)
    dst2d = jnp.concatenate(
        [dst, jnp.full((npad,), _N, jnp.int32)]).reshape(_EPAD // 128, 128)

    ones_x = jnp.ones((_N, 128), jnp.float32)
    degp = _prop_call(1)(ones_x, src2d, dst2d)[0].reshape(_NC, _NPAD, 128)
    norm, xsa, xsb = _norm_call(degp, x)
    xs = [xsa, xsb]

    h = x
    params = [(W1, b1), (W2, b2), (W3, b3), (W4, b4), (W5, b5), (W6, b6),
              (W7, b7)]
    for li, (W, b) in enumerate(params):
        nchunk = len(xs)
        prop = _prop_call(nchunk)
        aggs1 = [a.reshape(_NC, _NPAD, 128) for a in prop(*xs, src2d, dst2d)]
        r1s, xs2 = _mid_call(aggs1, norm)
        aggs2 = [a.reshape(_NC, _NPAD, 128) for a in prop(*xs2, src2d, dst2d)]
        h, xs = _layer_call(h, r1s, aggs2, W, b, norm,
                            emit_xs=(li < len(params) - 1))
    return _pool(h, Wc, bc)
